# baseline (speedup 1.0000x reference)
import functools

import jax
import jax.numpy as jnp
from jax import lax
from jax.experimental import pallas as pl
from jax.experimental.pallas import tpu as pltpu
from jax.experimental.pallas import tpu_sc as plsc

_N_STATES = 1000 * 1000
_N_ACTIONS = 16
_BATCH = 16384

_NC = 2
_NS = 16
_NW = _NC * _NS
_L = 16

_NB = _N_STATES // 128
_W = 2048
_WT = 256
_TAILB = _N_STATES - _WT
_NSUB = 16
_NVEC = _BATCH // _L

_sc_mesh = plsc.VectorSubcoreMesh(core_axis_name="c", subcore_axis_name="s")


@functools.partial(
    pl.kernel,
    mesh=_sc_mesh,
    out_type=(jax.ShapeDtypeStruct((_N_ACTIONS, _N_STATES), jnp.float32),
              jax.ShapeDtypeStruct((_N_ACTIONS, _WT), jnp.float32)),
    compiler_params=pltpu.CompilerParams(needs_layout_passes=False),
    scratch_types=[
        pltpu.VMEM((_BATCH,), jnp.int32),
        pltpu.VMEM((_BATCH,), jnp.float32),
        pltpu.VMEM((_BATCH + _L,), jnp.int32),
        pltpu.VMEM((_N_ACTIONS, _W), jnp.float32),
        pltpu.VMEM((_N_ACTIONS, _W), jnp.float32),
        pltpu.VMEM((_N_ACTIONS, _WT), jnp.float32),
        pltpu.VMEM((_L,), jnp.float32),
        pltpu.SemaphoreType.DMA,
        pltpu.SemaphoreType.DMA,
        pltpu.SemaphoreType.DMA,
        pltpu.SemaphoreType.DMA,
        pltpu.SemaphoreType.DMA,
    ],
)
def _sc_copy_update(qT_hbm, tail_hbm, key_hbm, tgt_hbm, lr_hbm,
                    outT_hbm, otail_hbm,
                    key_v, tgt_v, cid_v, buf_a, buf_b, buf_t, lr_v,
                    sem_ia, sem_ib, sem_oa, sem_ob, sem_x):
    wid = lax.axis_index("s") * _NC + lax.axis_index("c")
    is_last = wid == _NW - 1
    b_lane = (wid * _NB) // _NW * 128
    e_lane = ((wid + 1) * _NB) // _NW * 128

    def _start_in(j):
        s_j = jnp.minimum(b_lane + j * _W, e_lane - _W)
        buf = buf_a if j % 2 == 0 else buf_b
        sem = sem_ia if j % 2 == 0 else sem_ib
        cp = pltpu.async_copy(qT_hbm.at[:, pl.ds(s_j, _W)], buf, sem)
        return s_j, buf, cp

    pltpu.sync_copy(key_hbm, key_v)
    ins = [_start_in(0), _start_in(1)]
    cp_tgt = pltpu.async_copy(tgt_hbm, tgt_v, sem_x)
    cp_lr = pltpu.async_copy(lr_hbm, lr_v, sem_x)
    iota = lax.iota(jnp.int32, _L)

    b_key = b_lane * _N_ACTIONS
    e_key = jnp.where(is_last, _N_STATES, e_lane) * _N_ACTIONS

    @pl.loop(0, _NVEC, init_carry=jnp.int32(0), unroll=4)
    def _scan(i, cnt):
        k = key_v[pl.ds(i * _L, _L)]
        m = (k >= b_key) & (k < e_key)
        plsc.store_compressed(cid_v.at[pl.ds(cnt, _L)], iota + i * _L, mask=m)
        return cnt + jnp.max(plsc.all_reduce_population_count(m))

    n_upd = _scan
    cp_tgt.wait()
    cp_lr.wait()
    lr_b = lr_v[...]

    def _apply(buf, s_lane, width):
        nv = (n_upd + _L - 1) // _L
        s_key = s_lane * _N_ACTIONS

        @pl.loop(0, nv)
        def _inner(v):
            lanes = iota + v * _L
            lane_ok = lanes < n_upd
            e = cid_v[pl.ds(v * _L, _L)]
            e = jnp.where(lane_ok, e, 0)
            k = plsc.load_gather(key_v, [e])
            win = lane_ok & (k >= s_key) & (k < s_key + width * _N_ACTIONS)
            t = plsc.load_gather(tgt_v, [e])
            kz = jnp.where(win, k - s_key, 0)
            sloc = lax.shift_right_logical(kz, 4)
            a = lax.bitwise_and(kz, _N_ACTIONS - 1)
            cur = plsc.load_gather(buf, [a, sloc], mask=win)
            new = cur + lr_b * (t - cur)
            plsc.store_scatter(buf, [a, sloc], new, mask=win)

    outs = [None, None]
    for j in range(_NSUB):
        s_j, buf, cp_in = ins[j % 2]
        cp_in.wait()
        _apply(buf, s_j, _W)
        sem_o = sem_oa if j % 2 == 0 else sem_ob
        outs[j % 2] = pltpu.async_copy(buf, outT_hbm.at[:, pl.ds(s_j, _W)],
                                       sem_o)
        if j + 2 < _NSUB:
            outs[j % 2].wait()
            outs[j % 2] = None
            ins[j % 2] = _start_in(j + 2)

    @pl.when(is_last)
    def _tail():
        pltpu.async_copy(tail_hbm, buf_t, sem_x).wait()
        _apply(buf_t, _TAILB, _WT)
        pltpu.async_copy(buf_t, otail_hbm, sem_x).wait()

    for o in outs:
        if o is not None:
            o.wait()


def kernel(q_tables, pos, target_val, lr, act):
    lr16 = jnp.broadcast_to(lr, (_L,))
    keys = pos * _N_ACTIONS + act
    tail_in = lax.slice(q_tables.T, (0, _TAILB), (_N_ACTIONS, _N_STATES))
    outT, otail = _sc_copy_update(q_tables.T, tail_in, keys, target_val, lr16)
    outT = lax.dynamic_update_slice(outT, otail, (0, _TAILB))
    return outT.T


# speedup vs baseline: 1.0083x; 1.0083x over previous
import functools

import jax
import jax.numpy as jnp
from jax import lax
from jax.experimental import pallas as pl
from jax.experimental.pallas import tpu as pltpu
from jax.experimental.pallas import tpu_sc as plsc

_N_STATES = 1000 * 1000
_N_ACTIONS = 16
_BATCH = 16384

_NC = 2
_NS = 16
_NW = _NC * _NS
_L = 16

_NB = _N_STATES // 128
_W = 1792
_WT = 256
_TAILB = _N_STATES - _WT
_NSUB = 18
_NVEC = _BATCH // _L

_sc_mesh = plsc.VectorSubcoreMesh(core_axis_name="c", subcore_axis_name="s")


@functools.partial(
    pl.kernel,
    mesh=_sc_mesh,
    out_type=(jax.ShapeDtypeStruct((_N_ACTIONS, _N_STATES), jnp.float32),
              jax.ShapeDtypeStruct((_N_ACTIONS, _WT), jnp.float32)),
    compiler_params=pltpu.CompilerParams(needs_layout_passes=False),
    scratch_types=[
        pltpu.VMEM((_BATCH,), jnp.int32),
        pltpu.VMEM((_BATCH,), jnp.int32),
        pltpu.VMEM((_BATCH,), jnp.float32),
        pltpu.VMEM((_BATCH + _L,), jnp.int32),
        pltpu.VMEM((_N_ACTIONS, _W), jnp.float32),
        pltpu.VMEM((_N_ACTIONS, _W), jnp.float32),
        pltpu.VMEM((_N_ACTIONS, _WT), jnp.float32),
        pltpu.VMEM((_L,), jnp.float32),
        pltpu.SemaphoreType.DMA,
        pltpu.SemaphoreType.DMA,
        pltpu.SemaphoreType.DMA,
        pltpu.SemaphoreType.DMA,
        pltpu.SemaphoreType.DMA,
    ],
)
def _sc_copy_update(qT_hbm, tail_hbm, pos_hbm, act_hbm, tgt_hbm, lr_hbm,
                    outT_hbm, otail_hbm,
                    pos_v, act_v, tgt_v, cid_v, buf_a, buf_b, buf_t, lr_v,
                    sem_ia, sem_ib, sem_oa, sem_ob, sem_x):
    wid = lax.axis_index("s") * _NC + lax.axis_index("c")
    is_last = wid == _NW - 1
    b_lane = (wid * _NB) // _NW * 128
    e_lane = ((wid + 1) * _NB) // _NW * 128

    def _start_in(j):
        s_j = jnp.minimum(b_lane + j * _W, e_lane - _W)
        buf = buf_a if j % 2 == 0 else buf_b
        sem = sem_ia if j % 2 == 0 else sem_ib
        cp = pltpu.async_copy(qT_hbm.at[:, pl.ds(s_j, _W)], buf, sem)
        return s_j, buf, cp

    pltpu.sync_copy(pos_hbm, pos_v)
    ins = [_start_in(0), _start_in(1)]
    cp_act = pltpu.async_copy(act_hbm, act_v, sem_x)
    cp_tgt = pltpu.async_copy(tgt_hbm, tgt_v, sem_x)
    cp_lr = pltpu.async_copy(lr_hbm, lr_v, sem_x)
    iota = lax.iota(jnp.int32, _L)

    e_scan = jnp.where(is_last, _N_STATES, e_lane)

    @pl.loop(0, _NVEC, init_carry=jnp.int32(0), unroll=4)
    def _scan(i, cnt):
        p = pos_v[pl.ds(i * _L, _L)]
        m = (p >= b_lane) & (p < e_scan)
        plsc.store_compressed(cid_v.at[pl.ds(cnt, _L)], iota + i * _L, mask=m)
        return cnt + jnp.max(plsc.all_reduce_population_count(m))

    n_upd = _scan
    cp_act.wait()
    cp_tgt.wait()
    cp_lr.wait()
    lr_b = lr_v[...]

    def _apply(buf, s_lane, width):
        nv = (n_upd + _L - 1) // _L

        @pl.loop(0, nv)
        def _inner(v):
            lanes = iota + v * _L
            lane_ok = lanes < n_upd
            e = cid_v[pl.ds(v * _L, _L)]
            e = jnp.where(lane_ok, e, 0)
            p = plsc.load_gather(pos_v, [e])
            win = lane_ok & (p >= s_lane) & (p < s_lane + width)
            a = plsc.load_gather(act_v, [e])
            t = plsc.load_gather(tgt_v, [e])
            sloc = jnp.where(win, p - s_lane, 0)
            a = jnp.where(win, a, 0)
            cur = plsc.load_gather(buf, [a, sloc], mask=win)
            new = cur + lr_b * (t - cur)
            plsc.store_scatter(buf, [a, sloc], new, mask=win)

    outs = [None, None]
    for j in range(_NSUB):
        s_j, buf, cp_in = ins[j % 2]
        cp_in.wait()
        _apply(buf, s_j, _W)
        sem_o = sem_oa if j % 2 == 0 else sem_ob
        outs[j % 2] = pltpu.async_copy(buf, outT_hbm.at[:, pl.ds(s_j, _W)],
                                       sem_o)
        if j + 2 < _NSUB:
            outs[j % 2].wait()
            outs[j % 2] = None
            ins[j % 2] = _start_in(j + 2)

    @pl.when(is_last)
    def _tail():
        pltpu.async_copy(tail_hbm, buf_t, sem_x).wait()
        _apply(buf_t, _TAILB, _WT)
        pltpu.async_copy(buf_t, otail_hbm, sem_x).wait()

    for o in outs:
        if o is not None:
            o.wait()


def kernel(q_tables, pos, target_val, lr, act):
    lr16 = jnp.broadcast_to(lr, (_L,))
    tail_in = lax.slice(q_tables.T, (0, _TAILB), (_N_ACTIONS, _N_STATES))
    outT, otail = _sc_copy_update(q_tables.T, tail_in, pos, act, target_val,
                                  lr16)
    outT = lax.dynamic_update_slice(outT, otail, (0, _TAILB))
    return outT.T
